# baseline (speedup 1.0000x reference)
import functools

import jax
import jax.numpy as jnp
from jax import lax
from jax.experimental import pallas as pl
from jax.experimental.pallas import tpu as pltpu
from jax.experimental.pallas import tpu_sc as plsc

N, D = 262144, 256
H = D // 2
NUM_CORES = 2
NUM_SUBCORES = 16
NW = NUM_CORES * NUM_SUBCORES
ROWS_PER_W = N // NW
CHUNKS = 4
ROWS_PER_CHUNK = ROWS_PER_W // CHUNKS

_mesh = plsc.VectorSubcoreMesh(core_axis_name="c", subcore_axis_name="s")


@functools.partial(
    pl.kernel,
    mesh=_mesh,
    out_type=(
        jax.ShapeDtypeStruct((N, H), jnp.float32),
        jax.ShapeDtypeStruct((N, H), jnp.float32),
    ),
    scratch_types=[pltpu.SemaphoreType.DMA],
)
def _split_halves(inp_hbm, speed_hbm, dir_hbm, sem):
    wid = lax.axis_index("s") * NUM_CORES + lax.axis_index("c")
    base = wid * ROWS_PER_W
    copies = []
    for i in range(CHUNKS):
        rows = pl.ds(base + i * ROWS_PER_CHUNK, ROWS_PER_CHUNK)
        copies.append(pltpu.async_copy(inp_hbm.at[rows, pl.ds(0, H)], speed_hbm.at[rows], sem))
        copies.append(pltpu.async_copy(inp_hbm.at[rows, pl.ds(H, H)], dir_hbm.at[rows], sem))
    for c in copies:
        c.wait()


def kernel(inputs):
    return _split_halves(inputs)


# speedup vs baseline: 39.3443x; 39.3443x over previous
import functools

import jax
import jax.numpy as jnp
from jax import lax
from jax.experimental import pallas as pl
from jax.experimental.pallas import tpu as pltpu
from jax.experimental.pallas import tpu_sc as plsc

N, D = 262144, 256
H = D // 2
NUM_CORES = 2
NUM_SUBCORES = 16
NW = NUM_CORES * NUM_SUBCORES
ROWS_PER_W = N // NW
R = 128
CHUNKS = ROWS_PER_W // R
NBUF = 3

_mesh = plsc.VectorSubcoreMesh(core_axis_name="c", subcore_axis_name="s")


@functools.partial(
    pl.kernel,
    mesh=_mesh,
    out_type=(
        jax.ShapeDtypeStruct((N, H), jnp.float32),
        jax.ShapeDtypeStruct((N, H), jnp.float32),
    ),
    scratch_types=[
        pltpu.VMEM((NBUF, R, D), jnp.float32),
        pltpu.SemaphoreType.DMA,
        pltpu.SemaphoreType.DMA,
    ],
)
def _split_halves(inp_hbm, speed_hbm, dir_hbm, buf, in_sem, out_sem):
    wid = lax.axis_index("s") * NUM_CORES + lax.axis_index("c")
    base = wid * ROWS_PER_W

    def rows(i):
        return pl.ds(base + i * R, R)

    def start_read(i, slot):
        pltpu.async_copy(inp_hbm.at[rows(i)], buf.at[slot], in_sem)

    def wait_read(i, slot):
        pltpu.make_async_copy(inp_hbm.at[rows(i)], buf.at[slot], in_sem).wait()

    def start_writes(i, slot):
        pltpu.async_copy(buf.at[slot, :, pl.ds(0, H)], speed_hbm.at[rows(i)], out_sem)
        pltpu.async_copy(buf.at[slot, :, pl.ds(H, H)], dir_hbm.at[rows(i)], out_sem)

    def wait_writes(i, slot):
        pltpu.make_async_copy(buf.at[slot, :, pl.ds(0, H)], speed_hbm.at[rows(i)], out_sem).wait()
        pltpu.make_async_copy(buf.at[slot, :, pl.ds(H, H)], dir_hbm.at[rows(i)], out_sem).wait()

    for j in range(NBUF):
        start_read(j, j)

    def body(i, _):
        slot = lax.rem(i, NBUF)

        @pl.when(i >= 1)
        def _():
            prev_slot = lax.rem(i - 1, NBUF)
            wait_writes(i - 1, prev_slot)

            @pl.when(i - 1 + NBUF < CHUNKS)
            def _():
                start_read(i - 1 + NBUF, prev_slot)

        wait_read(i, slot)
        start_writes(i, slot)
        return 0

    lax.fori_loop(0, CHUNKS, body, 0)
    wait_writes(CHUNKS - 1, lax.rem(CHUNKS - 1, NBUF))


def kernel(inputs):
    return _split_halves(inputs)


# speedup vs baseline: 39.4633x; 1.0030x over previous
import functools

import jax
import jax.numpy as jnp
from jax import lax
from jax.experimental import pallas as pl
from jax.experimental.pallas import tpu as pltpu
from jax.experimental.pallas import tpu_sc as plsc

N, D = 262144, 256
H = D // 2
NUM_CORES = 2
NUM_SUBCORES = 16
NW = NUM_CORES * NUM_SUBCORES
ROWS_PER_W = N // NW
R = 64
CHUNKS = ROWS_PER_W // R
NBUF = 7

_mesh = plsc.VectorSubcoreMesh(core_axis_name="c", subcore_axis_name="s")


@functools.partial(
    pl.kernel,
    mesh=_mesh,
    out_type=(
        jax.ShapeDtypeStruct((N, H), jnp.float32),
        jax.ShapeDtypeStruct((N, H), jnp.float32),
    ),
    scratch_types=[
        pltpu.VMEM((NBUF, R, D), jnp.float32),
        pltpu.SemaphoreType.DMA,
        pltpu.SemaphoreType.DMA,
    ],
)
def _split_halves(inp_hbm, speed_hbm, dir_hbm, buf, in_sem, out_sem):
    wid = lax.axis_index("s") * NUM_CORES + lax.axis_index("c")
    base = wid * ROWS_PER_W

    def rows(i):
        return pl.ds(base + i * R, R)

    def start_read(i, slot):
        pltpu.async_copy(inp_hbm.at[rows(i)], buf.at[slot], in_sem)

    def wait_read(i, slot):
        pltpu.make_async_copy(inp_hbm.at[rows(i)], buf.at[slot], in_sem).wait()

    def start_writes(i, slot):
        pltpu.async_copy(buf.at[slot, :, pl.ds(0, H)], speed_hbm.at[rows(i)], out_sem)
        pltpu.async_copy(buf.at[slot, :, pl.ds(H, H)], dir_hbm.at[rows(i)], out_sem)

    def wait_writes(i, slot):
        pltpu.make_async_copy(buf.at[slot, :, pl.ds(0, H)], speed_hbm.at[rows(i)], out_sem).wait()
        pltpu.make_async_copy(buf.at[slot, :, pl.ds(H, H)], dir_hbm.at[rows(i)], out_sem).wait()

    for j in range(NBUF):
        start_read(j, j)

    def body(i, _):
        slot = lax.rem(i, NBUF)

        @pl.when(i >= 1)
        def _():
            prev_slot = lax.rem(i - 1, NBUF)
            wait_writes(i - 1, prev_slot)

            @pl.when(i - 1 + NBUF < CHUNKS)
            def _():
                start_read(i - 1 + NBUF, prev_slot)

        wait_read(i, slot)
        start_writes(i, slot)
        return 0

    lax.fori_loop(0, CHUNKS, body, 0)
    wait_writes(CHUNKS - 1, lax.rem(CHUNKS - 1, NBUF))


def kernel(inputs):
    return _split_halves(inputs)
